# baseline (speedup 1.0000x reference)
import functools

import jax
import jax.numpy as jnp
from jax import lax
from jax.experimental import pallas as pl
from jax.experimental.pallas import tpu as pltpu
from jax.experimental.pallas import tpu_sc as plsc

B = 4096
H = 20
E = 64
V = 100000
NTAB = 5
DENSE = 256
HIDDEN = 512

VP = 102400
Q = VP // 4
QBLK = 6400

NC, NS, L = 2, 16, 16
NW = NC * NS
BAGS_W = B // NW
CHUNK = 32
NCHUNK = BAGS_W // CHUNK
ROWS_CHUNK = CHUNK * H


def _sc_pool(idx, tab):
    mesh = plsc.VectorSubcoreMesh(core_axis_name="c", subcore_axis_name="s")

    @functools.partial(
        pl.kernel,
        out_type=jax.ShapeDtypeStruct((B, E), jnp.float32),
        mesh=mesh,
        scratch_types=[
            pltpu.VMEM((BAGS_W * H,), jnp.int32),
            pltpu.VMEM((2, ROWS_CHUNK, E // 2), jnp.int32),
            pltpu.VMEM((BAGS_W, E), jnp.float32),
            pltpu.SemaphoreType.DMA,
            pltpu.SemaphoreType.DMA,
            pltpu.SemaphoreType.DMA,
        ],
        compiler_params=pltpu.CompilerParams(use_tc_tiling_on_sc=False,
                                             needs_layout_passes=False),
    )
    def k(ihbm, thbm, out_hbm, idx_v, rows_v, pool_v, sem0, sem1, osem):
        wid = lax.axis_index("s") * NC + lax.axis_index("c")
        base_bag = wid * BAGS_W
        base_idx = base_bag * H
        sems = (sem0, sem1)
        pltpu.sync_copy(ihbm.at[pl.ds(base_idx, BAGS_W * H)], idx_v)

        def start(c, slot):
            return pltpu.async_copy(
                thbm.at[idx_v.at[pl.ds(c * ROWS_CHUNK, ROWS_CHUNK)]],
                rows_v.at[slot], sems[slot])

        handles = {0: start(0, 0)}
        for c in range(NCHUNK):
            slot = c % 2
            if c + 1 < NCHUNK:
                handles[c + 1] = start(c + 1, 1 - slot)
            handles.pop(c).wait()

            def body(bag, carry, _c=c, _slot=slot):
                r0 = bag * H
                for j in range(E // (2 * L)):
                    words = rows_v[_slot, r0, pl.ds(j * L, L)]
                    acc_a, acc_b = plsc.unpack(
                        plsc.bitcast(words, jnp.bfloat16),
                        format=plsc.PackFormat.INTERLEAVED)
                    for q in range(1, H):
                        words = rows_v[_slot, r0 + q, pl.ds(j * L, L)]
                        a, b = plsc.unpack(
                            plsc.bitcast(words, jnp.bfloat16),
                            format=plsc.PackFormat.INTERLEAVED)
                        acc_a = acc_a + a
                        acc_b = acc_b + b
                    pool_v[_c * CHUNK + bag, pl.ds(2 * j * L, L)] = acc_a
                    pool_v[_c * CHUNK + bag, pl.ds((2 * j + 1) * L, L)] = acc_b
                return carry

            lax.fori_loop(0, CHUNK, body, 0)
        pltpu.async_copy(pool_v, out_hbm.at[pl.ds(base_bag, BAGS_W)],
                         osem).wait()

    return k(idx, tab)


def _tc_repack(tT):
    def pack(x_ref):
        xt = jnp.swapaxes(x_ref[...], 0, 1)
        we = jax.lax.bitcast_convert_type(
            xt[:, 0:E // 2].astype(jnp.bfloat16), jnp.uint16).astype(jnp.uint32)
        wo = jax.lax.bitcast_convert_type(
            xt[:, E // 2:E].astype(jnp.bfloat16), jnp.uint16).astype(jnp.uint32)
        return jax.lax.bitcast_convert_type(we | (wo << 16), jnp.int32)

    def body(x1_ref, x2_ref, x3_ref, x4_ref, o_ref):
        for k, x_ref in enumerate((x1_ref, x2_ref, x3_ref, x4_ref)):
            o_ref[:, 32 * k:32 * (k + 1)] = pack(x_ref)

    nblk = Q // QBLK
    last = -(-V // QBLK) - 1

    def shifted(k):
        return lambda i: (0, jnp.minimum(i + k * (Q // QBLK), last))

    return pl.pallas_call(
        body,
        grid=(nblk,),
        in_specs=[
            pl.BlockSpec((E, QBLK), shifted(0)),
            pl.BlockSpec((E, QBLK), shifted(1)),
            pl.BlockSpec((E, QBLK), shifted(2)),
            pl.BlockSpec((E, QBLK), shifted(3)),
        ],
        out_specs=pl.BlockSpec((QBLK, 2 * E), lambda i: (i, 0)),
        out_shape=jax.ShapeDtypeStruct((Q, 2 * E), jnp.int32),
    )(tT, tT, tT, tT)


def _mlp(pooled, dense, W1, b1r, W2r, b2r):
    BLK = 512
    FUSED = NTAB * E + DENSE

    def body(p0, p1, p2, p3, p4, d_ref, w1_ref, b1_ref, w2_ref, b2_ref, o_ref):
        h = jnp.dot(d_ref[...], w1_ref[NTAB * E:, :],
                    preferred_element_type=jnp.float32)
        for t, p_ref in enumerate((p0, p1, p2, p3, p4)):
            h = h + jnp.dot(p_ref[...], w1_ref[t * E:(t + 1) * E, :],
                            preferred_element_type=jnp.float32)
        h = h + b1_ref[...]
        h = h * jax.nn.sigmoid(h)
        o_ref[...] = jnp.sum(h * w2_ref[...], axis=1, keepdims=True) + b2_ref[...]

    return pl.pallas_call(
        body,
        grid=(B // BLK,),
        in_specs=[pl.BlockSpec((BLK, E), lambda i: (i, 0))] * NTAB + [
            pl.BlockSpec((BLK, DENSE), lambda i: (i, 0)),
            pl.BlockSpec((FUSED, HIDDEN), lambda i: (0, 0)),
            pl.BlockSpec((1, HIDDEN), lambda i: (0, 0)),
            pl.BlockSpec((1, HIDDEN), lambda i: (0, 0)),
            pl.BlockSpec((1, 1), lambda i: (0, 0)),
        ],
        out_specs=pl.BlockSpec((BLK, 1), lambda i: (i, 0)),
        out_shape=jax.ShapeDtypeStruct((B, 1), jnp.float32),
    )(*pooled, dense, W1, b1r, W2r, b2r)


def kernel(user_tokens, context_tokens, candidate_tokens,
           candidate_post_tokens, candidate_author_tokens, dense_features,
           table_user_tokens, table_context_tokens, table_candidate_tokens,
           table_candidate_post_tokens, table_candidate_author_tokens,
           W1, b1, W2, b2):
    idx = []
    for t in (user_tokens, context_tokens, candidate_tokens,
              candidate_post_tokens, candidate_author_tokens):
        t = t.astype(jnp.int32)
        t = (t % Q) * 4 + t // Q
        idx.append(jnp.reshape(t, (B * H,)))
    pooled = []
    for i, tbl in enumerate((table_user_tokens, table_context_tokens,
                             table_candidate_tokens,
                             table_candidate_post_tokens,
                             table_candidate_author_tokens)):
        tab = jnp.reshape(_tc_repack(tbl.T), (VP, E // 2))
        pooled.append(_sc_pool(idx[i], tab))
    perm = (list(range(0, 16)) + list(range(32, 48))
            + list(range(16, 32)) + list(range(48, 64)))
    w1p = jnp.concatenate(
        [W1[t * E:(t + 1) * E][jnp.array(perm)] for t in range(NTAB)]
        + [W1[NTAB * E:]], axis=0)
    out = _mlp(pooled, dense_features, w1p,
               jnp.reshape(b1, (1, HIDDEN)),
               jnp.reshape(W2, (1, HIDDEN)),
               jnp.reshape(b2, (1, 1)))
    return jnp.squeeze(out, axis=-1)


# speedup vs baseline: 1.0234x; 1.0234x over previous
import functools

import jax
import jax.numpy as jnp
from jax import lax
from jax.experimental import pallas as pl
from jax.experimental.pallas import tpu as pltpu
from jax.experimental.pallas import tpu_sc as plsc

B = 4096
H = 20
E = 64
V = 100000
NTAB = 5
DENSE = 256
HIDDEN = 512

VP = 102400
Q = VP // 4
QBLK = 3200

NC, NS, L = 2, 16, 16
NW = NC * NS
BAGS_W = B // NW
CHUNK = 32
NCHUNK = BAGS_W // CHUNK
ROWS_CHUNK = CHUNK * H


def _sc_pool(idx, tab):
    mesh = plsc.VectorSubcoreMesh(core_axis_name="c", subcore_axis_name="s")

    @functools.partial(
        pl.kernel,
        out_type=jax.ShapeDtypeStruct((B, E), jnp.float32),
        mesh=mesh,
        scratch_types=[
            pltpu.VMEM((BAGS_W * H,), jnp.int32),
            pltpu.VMEM((2, ROWS_CHUNK, E // 2), jnp.int32),
            pltpu.VMEM((BAGS_W, E), jnp.float32),
            pltpu.SemaphoreType.DMA,
            pltpu.SemaphoreType.DMA,
            pltpu.SemaphoreType.DMA,
        ],
        compiler_params=pltpu.CompilerParams(use_tc_tiling_on_sc=False,
                                             needs_layout_passes=False),
    )
    def k(ihbm, thbm, out_hbm, idx_v, rows_v, pool_v, sem0, sem1, osem):
        wid = lax.axis_index("s") * NC + lax.axis_index("c")
        base_bag = wid * BAGS_W
        base_idx = base_bag * H
        sems = (sem0, sem1)
        pltpu.sync_copy(ihbm.at[pl.ds(base_idx, BAGS_W * H)], idx_v)

        def start(c, slot):
            return pltpu.async_copy(
                thbm.at[idx_v.at[pl.ds(c * ROWS_CHUNK, ROWS_CHUNK)]],
                rows_v.at[slot], sems[slot])

        handles = {0: start(0, 0)}
        for c in range(NCHUNK):
            slot = c % 2
            if c + 1 < NCHUNK:
                handles[c + 1] = start(c + 1, 1 - slot)
            handles.pop(c).wait()

            def body(bag, carry, _c=c, _slot=slot):
                r0 = bag * H
                for j in range(E // (2 * L)):
                    words = rows_v[_slot, r0, pl.ds(j * L, L)]
                    acc_a, acc_b = plsc.unpack(
                        plsc.bitcast(words, jnp.bfloat16),
                        format=plsc.PackFormat.INTERLEAVED)
                    for q in range(1, H):
                        words = rows_v[_slot, r0 + q, pl.ds(j * L, L)]
                        a, b = plsc.unpack(
                            plsc.bitcast(words, jnp.bfloat16),
                            format=plsc.PackFormat.INTERLEAVED)
                        acc_a = acc_a + a
                        acc_b = acc_b + b
                    pool_v[_c * CHUNK + bag, pl.ds(2 * j * L, L)] = acc_a
                    pool_v[_c * CHUNK + bag, pl.ds((2 * j + 1) * L, L)] = acc_b
                return carry

            lax.fori_loop(0, CHUNK, body, 0)
        pltpu.async_copy(pool_v, out_hbm.at[pl.ds(base_bag, BAGS_W)],
                         osem).wait()

    return k(idx, tab)


def _tc_repack(tT):
    def pack(x_ref):
        xt = jnp.swapaxes(x_ref[...], 0, 1)
        we = jax.lax.bitcast_convert_type(
            xt[:, 0:E // 2].astype(jnp.bfloat16), jnp.uint16).astype(jnp.uint32)
        wo = jax.lax.bitcast_convert_type(
            xt[:, E // 2:E].astype(jnp.bfloat16), jnp.uint16).astype(jnp.uint32)
        return jax.lax.bitcast_convert_type(we | (wo << 16), jnp.int32)

    def body(x1_ref, x2_ref, x3_ref, x4_ref, o_ref):
        for k, x_ref in enumerate((x1_ref, x2_ref, x3_ref, x4_ref)):
            o_ref[:, 32 * k:32 * (k + 1)] = pack(x_ref)

    nblk = Q // QBLK
    last = -(-V // QBLK) - 1

    def shifted(k):
        return lambda i: (0, jnp.minimum(i + k * (Q // QBLK), last))

    return pl.pallas_call(
        body,
        grid=(nblk,),
        in_specs=[
            pl.BlockSpec((E, QBLK), shifted(0)),
            pl.BlockSpec((E, QBLK), shifted(1)),
            pl.BlockSpec((E, QBLK), shifted(2)),
            pl.BlockSpec((E, QBLK), shifted(3)),
        ],
        out_specs=pl.BlockSpec((QBLK, 2 * E), lambda i: (i, 0)),
        out_shape=jax.ShapeDtypeStruct((Q, 2 * E), jnp.int32),
    )(tT, tT, tT, tT)


def _mlp(pooled, dense, W1, b1r, W2r, b2r):
    BLK = 512
    FUSED = NTAB * E + DENSE

    def body(p0, p1, p2, p3, p4, d_ref, w1_ref, b1_ref, w2_ref, b2_ref, o_ref):
        h = jnp.dot(d_ref[...], w1_ref[NTAB * E:, :],
                    preferred_element_type=jnp.float32)
        for t, p_ref in enumerate((p0, p1, p2, p3, p4)):
            h = h + jnp.dot(p_ref[...], w1_ref[t * E:(t + 1) * E, :],
                            preferred_element_type=jnp.float32)
        h = h + b1_ref[...]
        h = h * jax.nn.sigmoid(h)
        o_ref[...] = jnp.sum(h * w2_ref[...], axis=1, keepdims=True) + b2_ref[...]

    return pl.pallas_call(
        body,
        grid=(B // BLK,),
        in_specs=[pl.BlockSpec((BLK, E), lambda i: (i, 0))] * NTAB + [
            pl.BlockSpec((BLK, DENSE), lambda i: (i, 0)),
            pl.BlockSpec((FUSED, HIDDEN), lambda i: (0, 0)),
            pl.BlockSpec((1, HIDDEN), lambda i: (0, 0)),
            pl.BlockSpec((1, HIDDEN), lambda i: (0, 0)),
            pl.BlockSpec((1, 1), lambda i: (0, 0)),
        ],
        out_specs=pl.BlockSpec((BLK, 1), lambda i: (i, 0)),
        out_shape=jax.ShapeDtypeStruct((B, 1), jnp.float32),
    )(*pooled, dense, W1, b1r, W2r, b2r)


def kernel(user_tokens, context_tokens, candidate_tokens,
           candidate_post_tokens, candidate_author_tokens, dense_features,
           table_user_tokens, table_context_tokens, table_candidate_tokens,
           table_candidate_post_tokens, table_candidate_author_tokens,
           W1, b1, W2, b2):
    idx = []
    for t in (user_tokens, context_tokens, candidate_tokens,
              candidate_post_tokens, candidate_author_tokens):
        t = t.astype(jnp.int32)
        t = (t % Q) * 4 + t // Q
        idx.append(jnp.reshape(t, (B * H,)))
    pooled = []
    for i, tbl in enumerate((table_user_tokens, table_context_tokens,
                             table_candidate_tokens,
                             table_candidate_post_tokens,
                             table_candidate_author_tokens)):
        tab = jnp.reshape(_tc_repack(tbl.T), (VP, E // 2))
        pooled.append(_sc_pool(idx[i], tab))
    perm = (list(range(0, 16)) + list(range(32, 48))
            + list(range(16, 32)) + list(range(48, 64)))
    w1p = jnp.concatenate(
        [W1[t * E:(t + 1) * E][jnp.array(perm)] for t in range(NTAB)]
        + [W1[NTAB * E:]], axis=0)
    out = _mlp(pooled, dense_features, w1p,
               jnp.reshape(b1, (1, HIDDEN)),
               jnp.reshape(W2, (1, HIDDEN)),
               jnp.reshape(b2, (1, 1)))
    return jnp.squeeze(out, axis=-1)
